# baseline (speedup 1.0000x reference)
import jax
import jax.numpy as jnp
import numpy as np
from jax.experimental import pallas as pl
from jax.experimental.pallas import tpu as pltpu

_B, _C, _H, _W = 8, 384, 32, 32
_MID, _GROUPS = 96, 8
_P = _H * _W
_KEEP = 256
_PAD = 64
_INT_MIN = np.int32(-2147483648)
_SQRT1_2 = np.float32(1.0 / np.sqrt(2.0))


def _gelu(x):
    return 0.5 * x * (1.0 + jax.lax.erf(x * _SQRT1_2))


def _dot_t(a, b):
    return jax.lax.dot_general(a, b, (((1,), (1,)), ((), ())),
                               preferred_element_type=jnp.float32)


def _body(en_ref, x_ref, w1_ref, b1_ref, w2_ref, b2_ref, w3_ref, b3_ref,
          mask_ref, scores_ref):
    en = en_ref[0] != 0
    ones_row = jnp.ones((1, _P), jnp.float32)

    def _count(ind_bool):
        ind = jnp.where(ind_bool, 1.0, 0.0)
        return jax.lax.dot_general(ones_row, ind, (((1,), (0,)), ((), ())),
                                   preferred_element_type=jnp.float32)

    X = x_ref[0]
    H1 = _gelu(jax.lax.dot_general(X, w1_ref[...], (((0,), (1,)), ((), ())),
                                   preferred_element_type=jnp.float32)
               + b1_ref[...])
    zpad = jnp.zeros((_PAD, _MID), jnp.float32)
    H1p = jnp.concatenate([zpad, H1, zpad], axis=0)
    xrow = jax.lax.broadcasted_iota(jnp.int32, (_P, 1), 0) % _W
    acc = b2_ref[...] + jnp.zeros((_P, _MID), jnp.float32)
    for dy in (-1, 0, 1):
        for dx in (-1, 0, 1):
            s = dy * _W + dx
            sh = jax.lax.slice(H1p, (_PAD + s, 0), (_PAD + s + _P, _MID))
            if dx == 1:
                sh = jnp.where(xrow != (_W - 1), sh, 0.0)
            elif dx == -1:
                sh = jnp.where(xrow != 0, sh, 0.0)
            t = (dy + 1) * 3 + (dx + 1)
            acc = acc + _dot_t(sh, w2_ref[t])
    H2 = _gelu(acc)
    S = _dot_t(H2, w3_ref[...]) + b3_ref[...]
    scores_ref[0] = jnp.where(en, jnp.transpose(S, (1, 0)), 0.0)

    bits = jax.lax.bitcast_convert_type(S, jnp.int32)
    keys = jnp.where(bits >= 0, bits, bits ^ np.int32(0x7FFFFFFF))

    lo0 = jnp.min(keys, axis=0, keepdims=True) - np.int32(1)
    hi0 = jnp.max(keys, axis=0, keepdims=True)
    kf = np.float32(_KEEP)

    def w_cond(state):
        it, lo, hi, T, done = state
        return (it < np.int32(34)) & (jnp.sum(done) < np.int32(_C))

    def w_body(state):
        it, lo, hi, T, done = state
        mid = (lo >> 1) + (hi >> 1) + (lo & hi & np.int32(1))
        cnt = _count(keys <= mid)
        hit = (cnt == kf) & (done == 0)
        T = jnp.where(hit, mid, T)
        done = jnp.where(hit, np.int32(1), done)
        ge = cnt >= kf
        hi = jnp.where(ge & (done == 0), mid, hi)
        lo = jnp.where((~ge) & (done == 0), mid + np.int32(1), lo)
        return (it + np.int32(1), lo, hi, T, done)

    zi = jnp.zeros((1, _C), jnp.int32)
    _, lo_f, _, T, done = jax.lax.while_loop(
        w_cond, w_body, (np.int32(0), lo0, hi0, zi, zi))
    Tk = jnp.where(done == 1, T, lo_f)
    le = keys <= Tk
    cnt_le = _count(le)
    any_tie = jnp.sum(jnp.where(cnt_le != kf, 1, 0)) > 0

    def no_tie():
        return le.astype(jnp.float32)

    def tie_fix():
        lt = keys < Tk
        eq = keys == Tk
        g = jnp.sum(lt.astype(jnp.int32), axis=0, keepdims=True)
        c = eq.astype(jnp.int32)
        shift = 1
        while shift < _P:
            c = c + jnp.concatenate(
                [jnp.zeros((shift, _C), jnp.int32),
                 jax.lax.slice(c, (0, 0), (_P - shift, _C))], axis=0)
            shift *= 2
        return (lt | (eq & (c <= (_KEEP - g)))).astype(jnp.float32)

    maskT = jax.lax.cond(any_tie, tie_fix, no_tie)
    mask_ref[0] = jnp.where(en, jnp.transpose(maskT, (1, 0)), 1.0)


def _block_diag_w2(w2):
    cpg = _MID // _GROUPS
    take = jnp.take(w2, jnp.asarray(np.arange(_MID) % cpg), axis=1)
    gi = np.arange(_MID) // cpg
    gmask = jnp.asarray((gi[:, None] == gi[None, :]).astype(np.float32))
    full = take * gmask[:, :, None, None]
    return jnp.transpose(full, (2, 3, 0, 1)).reshape(9, _MID, _MID)


def _run(en, xf, w1, b1, w2bd, b2, w3, b3):
    return pl.pallas_call(
        _body,
        grid=(_B,),
        in_specs=[
            pl.BlockSpec(memory_space=pltpu.SMEM),
            pl.BlockSpec((1, _C, _P), lambda b: (b, 0, 0)),
            pl.BlockSpec((_MID, _C), lambda b: (0, 0)),
            pl.BlockSpec((1, _MID), lambda b: (0, 0)),
            pl.BlockSpec((9, _MID, _MID), lambda b: (0, 0, 0)),
            pl.BlockSpec((1, _MID), lambda b: (0, 0)),
            pl.BlockSpec((_C, _MID), lambda b: (0, 0)),
            pl.BlockSpec((1, _C), lambda b: (0, 0)),
        ],
        out_specs=[
            pl.BlockSpec((1, _C, _P), lambda b: (b, 0, 0)),
            pl.BlockSpec((1, _C, _P), lambda b: (b, 0, 0)),
        ],
        out_shape=[
            jax.ShapeDtypeStruct((_B, _C, _P), jnp.float32),
            jax.ShapeDtypeStruct((_B, _C, _P), jnp.float32),
        ],
    )(en, xf, w1, b1, w2bd, b2, w3, b3)


def kernel(features, enabled, w1, b1, w2, b2, w3, b3):
    xf = features.reshape(_B, _C, _P)
    mask, scores = _run(
        jnp.asarray(enabled, jnp.int32).reshape(1),
        xf,
        w1.reshape(_MID, _C),
        b1.reshape(1, _MID),
        _block_diag_w2(w2),
        b2.reshape(1, _MID),
        w3.reshape(_C, _MID),
        b3.reshape(1, _C),
    )
    return (mask.reshape(_B, _C, _H, _W), scores.reshape(_B, _C, _H, _W))


# speedup vs baseline: 1.0822x; 1.0822x over previous
import jax
import jax.numpy as jnp
import numpy as np
from jax.experimental import pallas as pl
from jax.experimental.pallas import tpu as pltpu

_B, _C, _H, _W = 8, 384, 32, 32
_MID, _GROUPS = 96, 8
_P = _H * _W
_KEEP = 256
_PAD = 64
_INT_MIN = np.int32(-2147483648)
_SQRT1_2 = np.float32(1.0 / np.sqrt(2.0))


def _gelu(x):
    return 0.5 * x * (1.0 + jax.lax.erf(x * _SQRT1_2))


def _dot_t(a, b):
    return jax.lax.dot_general(a, b, (((1,), (1,)), ((), ())),
                               preferred_element_type=jnp.float32)


def _body(en_ref, x_ref, w1_ref, b1_ref, w2_ref, b2_ref, w3_ref, b3_ref,
          mask_ref, scores_ref):
    en = en_ref[0] != 0
    ones_row = jnp.ones((1, _P), jnp.float32)

    def _count(ind_bool):
        ind = jnp.where(ind_bool, 1.0, 0.0)
        return jax.lax.dot_general(ones_row, ind, (((1,), (0,)), ((), ())),
                                   preferred_element_type=jnp.float32)

    X = x_ref[0]
    H1 = _gelu(jax.lax.dot_general(X, w1_ref[...], (((0,), (1,)), ((), ())),
                                   preferred_element_type=jnp.float32)
               + b1_ref[...])
    zpad = jnp.zeros((_PAD, _MID), jnp.float32)
    H1p = jnp.concatenate([zpad, H1, zpad], axis=0)
    xrow = jax.lax.broadcasted_iota(jnp.int32, (_P, 1), 0) % _W
    acc = b2_ref[...] + jnp.zeros((_P, _MID), jnp.float32)
    for dy in (-1, 0, 1):
        for dx in (-1, 0, 1):
            s = dy * _W + dx
            sh = jax.lax.slice(H1p, (_PAD + s, 0), (_PAD + s + _P, _MID))
            if dx == 1:
                sh = jnp.where(xrow != (_W - 1), sh, 0.0)
            elif dx == -1:
                sh = jnp.where(xrow != 0, sh, 0.0)
            t = (dy + 1) * 3 + (dx + 1)
            acc = acc + _dot_t(sh, w2_ref[t])
    H2 = _gelu(acc)
    S = _dot_t(H2, w3_ref[...]) + b3_ref[...]
    scores_ref[0] = jnp.where(en, jnp.transpose(S, (1, 0)), 0.0)

    bits = jax.lax.bitcast_convert_type(S, jnp.int32)
    keys = jnp.where(bits >= 0, bits, bits ^ np.int32(0x7FFFFFFF))

    kf = np.float32(_KEEP)
    T = jnp.zeros((1, _C), jnp.int32)
    for i in range(32):
        cand_u = T | (np.int32(1) << (31 - i))
        cand_k = cand_u ^ _INT_MIN
        cnt = _count(keys < cand_k)
        T = jnp.where(cnt < kf, cand_u, T)
    Tk = T ^ _INT_MIN
    le = keys <= Tk
    cnt_le = _count(le)
    any_tie = jnp.sum(jnp.where(cnt_le != kf, 1, 0)) > 0

    def no_tie():
        return le.astype(jnp.float32)

    def tie_fix():
        lt = keys < Tk
        eq = keys == Tk
        g = jnp.sum(lt.astype(jnp.int32), axis=0, keepdims=True)
        c = eq.astype(jnp.int32)
        shift = 1
        while shift < _P:
            c = c + jnp.concatenate(
                [jnp.zeros((shift, _C), jnp.int32),
                 jax.lax.slice(c, (0, 0), (_P - shift, _C))], axis=0)
            shift *= 2
        return (lt | (eq & (c <= (_KEEP - g)))).astype(jnp.float32)

    maskT = jax.lax.cond(any_tie, tie_fix, no_tie)
    mask_ref[0] = jnp.where(en, jnp.transpose(maskT, (1, 0)), 1.0)


def _block_diag_w2(w2):
    cpg = _MID // _GROUPS
    take = jnp.take(w2, jnp.asarray(np.arange(_MID) % cpg), axis=1)
    gi = np.arange(_MID) // cpg
    gmask = jnp.asarray((gi[:, None] == gi[None, :]).astype(np.float32))
    full = take * gmask[:, :, None, None]
    return jnp.transpose(full, (2, 3, 0, 1)).reshape(9, _MID, _MID)


def _run(en, xf, w1, b1, w2bd, b2, w3, b3):
    return pl.pallas_call(
        _body,
        grid=(_B,),
        in_specs=[
            pl.BlockSpec(memory_space=pltpu.SMEM),
            pl.BlockSpec((1, _C, _P), lambda b: (b, 0, 0)),
            pl.BlockSpec((_MID, _C), lambda b: (0, 0)),
            pl.BlockSpec((1, _MID), lambda b: (0, 0)),
            pl.BlockSpec((9, _MID, _MID), lambda b: (0, 0, 0)),
            pl.BlockSpec((1, _MID), lambda b: (0, 0)),
            pl.BlockSpec((_C, _MID), lambda b: (0, 0)),
            pl.BlockSpec((1, _C), lambda b: (0, 0)),
        ],
        out_specs=[
            pl.BlockSpec((1, _C, _P), lambda b: (b, 0, 0)),
            pl.BlockSpec((1, _C, _P), lambda b: (b, 0, 0)),
        ],
        out_shape=[
            jax.ShapeDtypeStruct((_B, _C, _P), jnp.float32),
            jax.ShapeDtypeStruct((_B, _C, _P), jnp.float32),
        ],
    )(en, xf, w1, b1, w2bd, b2, w3, b3)


def kernel(features, enabled, w1, b1, w2, b2, w3, b3):
    xf = features.reshape(_B, _C, _P)
    mask, scores = _run(
        jnp.asarray(enabled, jnp.int32).reshape(1),
        xf,
        w1.reshape(_MID, _C),
        b1.reshape(1, _MID),
        _block_diag_w2(w2),
        b2.reshape(1, _MID),
        w3.reshape(_C, _MID),
        b3.reshape(1, _C),
    )
    return (mask.reshape(_B, _C, _H, _W), scores.reshape(_B, _C, _H, _W))


# speedup vs baseline: 1.0939x; 1.0108x over previous
import jax
import jax.numpy as jnp
import numpy as np
from jax.experimental import pallas as pl
from jax.experimental.pallas import tpu as pltpu

_B, _C, _H, _W = 8, 384, 32, 32
_MID, _GROUPS = 96, 8
_P = _H * _W
_KEEP = 256
_PAD = 64
_INT_MIN = np.int32(-2147483648)
_SQRT1_2 = np.float32(1.0 / np.sqrt(2.0))


def _gelu(x):
    return 0.5 * x * (1.0 + jax.lax.erf(x * _SQRT1_2))


def _dot_t(a, b):
    return jax.lax.dot_general(a, b, (((1,), (1,)), ((), ())),
                               preferred_element_type=jnp.float32)


def _body(en_ref, x_ref, w1_ref, b1_ref, w2_ref, b2_ref, w3_ref, b3_ref,
          mask_ref, scores_ref):
    en = en_ref[0] != 0
    ones_row = jnp.ones((1, _P), jnp.float32)

    def _count(ind_bool):
        ind = jnp.where(ind_bool, 1.0, 0.0)
        return jax.lax.dot_general(ones_row, ind, (((1,), (0,)), ((), ())),
                                   preferred_element_type=jnp.float32)

    X = x_ref[0]
    H1 = _gelu(jax.lax.dot_general(X, w1_ref[...], (((0,), (1,)), ((), ())),
                                   preferred_element_type=jnp.float32)
               + b1_ref[...])
    zpad = jnp.zeros((_PAD, _MID), jnp.float32)
    H1p = jnp.concatenate([zpad, H1, zpad], axis=0)
    xrow = jax.lax.broadcasted_iota(jnp.int32, (_P, 1), 0) % _W
    acc = b2_ref[...] + jnp.zeros((_P, _MID), jnp.float32)
    for dy in (-1, 0, 1):
        for dx in (-1, 0, 1):
            s = dy * _W + dx
            sh = jax.lax.slice(H1p, (_PAD + s, 0), (_PAD + s + _P, _MID))
            if dx == 1:
                sh = jnp.where(xrow != (_W - 1), sh, 0.0)
            elif dx == -1:
                sh = jnp.where(xrow != 0, sh, 0.0)
            t = (dy + 1) * 3 + (dx + 1)
            acc = acc + _dot_t(sh, w2_ref[t])
    H2 = _gelu(acc)
    S = _dot_t(H2, w3_ref[...]) + b3_ref[...]
    scores_ref[0] = jnp.where(en, jnp.transpose(S, (1, 0)), 0.0)

    bits = jax.lax.bitcast_convert_type(S, jnp.int32)
    keys = jnp.where(bits >= 0, bits, bits ^ np.int32(0x7FFFFFFF))

    kf = np.float32(_KEEP)
    ones_bf = jnp.ones((1, _P), jnp.bfloat16)

    def _count16(ind_bf):
        return jax.lax.dot_general(ones_bf, ind_bf, (((1,), (0,)), ((), ())),
                                   preferred_element_type=jnp.float32)

    hi16 = (keys >> 16).astype(jnp.int16)
    i16min = np.int32(-32768)

    def step_hi(i, T):
        cand_u = T | (np.int32(1) << (15 - i))
        cand_k = (cand_u ^ i16min).astype(jnp.int16)
        ind = jnp.where(hi16 < cand_k, jnp.bfloat16(1), jnp.bfloat16(0))
        return jnp.where(_count16(ind) < kf, cand_u, T)

    Tu = jax.lax.fori_loop(0, 16, step_hi, jnp.zeros((1, _C), jnp.int32))
    q = Tu ^ i16min
    q16 = q.astype(jnp.int16)
    g_hi = _count16(jnp.where(hi16 < q16, jnp.bfloat16(1), jnp.bfloat16(0)))
    r2 = kf - g_hi
    lo16 = ((keys & np.int32(0xFFFF)) + i16min).astype(jnp.int16)
    act = jnp.where(hi16 == q16, jnp.bfloat16(1), jnp.bfloat16(0))

    def step_lo(i, T):
        cand_u = T | (np.int32(1) << (15 - i))
        cand_k = (cand_u ^ i16min).astype(jnp.int16)
        ind = jnp.where(lo16 < cand_k, act, jnp.bfloat16(0))
        return jnp.where(_count16(ind) < r2, cand_u, T)

    Tlu = jax.lax.fori_loop(0, 16, step_lo, jnp.zeros((1, _C), jnp.int32))
    Tk = (q << 16) | (Tlu & np.int32(0xFFFF))
    le = keys <= Tk
    cnt_le = _count(le)
    any_tie = jnp.sum(jnp.where(cnt_le != kf, 1, 0)) > 0

    def no_tie():
        return le.astype(jnp.float32)

    def tie_fix():
        lt = keys < Tk
        eq = keys == Tk
        g = jnp.sum(lt.astype(jnp.int32), axis=0, keepdims=True)
        c = eq.astype(jnp.int32)
        shift = 1
        while shift < _P:
            c = c + jnp.concatenate(
                [jnp.zeros((shift, _C), jnp.int32),
                 jax.lax.slice(c, (0, 0), (_P - shift, _C))], axis=0)
            shift *= 2
        return (lt | (eq & (c <= (_KEEP - g)))).astype(jnp.float32)

    maskT = jax.lax.cond(any_tie, tie_fix, no_tie)
    mask_ref[0] = jnp.where(en, jnp.transpose(maskT, (1, 0)), 1.0)


def _block_diag_w2(w2):
    cpg = _MID // _GROUPS
    take = jnp.take(w2, jnp.asarray(np.arange(_MID) % cpg), axis=1)
    gi = np.arange(_MID) // cpg
    gmask = jnp.asarray((gi[:, None] == gi[None, :]).astype(np.float32))
    full = take * gmask[:, :, None, None]
    return jnp.transpose(full, (2, 3, 0, 1)).reshape(9, _MID, _MID)


def _run(en, xf, w1, b1, w2bd, b2, w3, b3):
    return pl.pallas_call(
        _body,
        grid=(_B,),
        in_specs=[
            pl.BlockSpec(memory_space=pltpu.SMEM),
            pl.BlockSpec((1, _C, _P), lambda b: (b, 0, 0)),
            pl.BlockSpec((_MID, _C), lambda b: (0, 0)),
            pl.BlockSpec((1, _MID), lambda b: (0, 0)),
            pl.BlockSpec((9, _MID, _MID), lambda b: (0, 0, 0)),
            pl.BlockSpec((1, _MID), lambda b: (0, 0)),
            pl.BlockSpec((_C, _MID), lambda b: (0, 0)),
            pl.BlockSpec((1, _C), lambda b: (0, 0)),
        ],
        out_specs=[
            pl.BlockSpec((1, _C, _P), lambda b: (b, 0, 0)),
            pl.BlockSpec((1, _C, _P), lambda b: (b, 0, 0)),
        ],
        out_shape=[
            jax.ShapeDtypeStruct((_B, _C, _P), jnp.float32),
            jax.ShapeDtypeStruct((_B, _C, _P), jnp.float32),
        ],
    )(en, xf, w1, b1, w2bd, b2, w3, b3)


def kernel(features, enabled, w1, b1, w2, b2, w3, b3):
    xf = features.reshape(_B, _C, _P)
    mask, scores = _run(
        jnp.asarray(enabled, jnp.int32).reshape(1),
        xf,
        w1.reshape(_MID, _C),
        b1.reshape(1, _MID),
        _block_diag_w2(w2),
        b2.reshape(1, _MID),
        w3.reshape(_C, _MID),
        b3.reshape(1, _C),
    )
    return (mask.reshape(_B, _C, _H, _W), scores.reshape(_B, _C, _H, _W))


# speedup vs baseline: 1.3701x; 1.2525x over previous
import jax
import jax.numpy as jnp
import numpy as np
from jax.experimental import pallas as pl
from jax.experimental.pallas import tpu as pltpu

_B, _C, _H, _W = 8, 384, 32, 32
_MID, _GROUPS = 96, 8
_P = _H * _W
_KEEP = 256
_PAD = 64
_NB = 4
_CW = _NB * _C
_PP = _NB * _P
_INT_MIN = np.int32(-2147483648)
_SQRT1_2 = np.float32(1.0 / np.sqrt(2.0))


def _gelu(x):
    return 0.5 * x * (1.0 + jax.lax.erf(x * _SQRT1_2))


def _dot_t(a, b):
    return jax.lax.dot_general(a, b, (((1,), (1,)), ((), ())),
                               preferred_element_type=jnp.float32)


def _body(en_ref, x_ref, w1_ref, b1_ref, w2_ref, b2_ref, w3_ref, b3_ref,
          mask_ref, scores_ref):
    en = en_ref[0] != 0

    h1s = []
    for bb in range(_NB):
        X = x_ref[bb]
        h1s.append(_gelu(
            jax.lax.dot_general(X, w1_ref[...], (((0,), (1,)), ((), ())),
                                preferred_element_type=jnp.float32)
            + b1_ref[...]))
    zpad = jnp.zeros((_PAD, _MID), jnp.float32)
    H1p = jnp.concatenate([zpad] + h1s + [zpad], axis=0)
    pos = jax.lax.broadcasted_iota(jnp.int32, (_PP, 1), 0)
    xc = pos % _W
    yc = (pos // _W) % _H
    acc = b2_ref[...] + jnp.zeros((_PP, _MID), jnp.float32)
    for dy in (-1, 0, 1):
        for dx in (-1, 0, 1):
            s = dy * _W + dx
            sh = jax.lax.slice(H1p, (_PAD + s, 0), (_PAD + s + _PP, _MID))
            if dx == 1:
                sh = jnp.where(xc != (_W - 1), sh, 0.0)
            elif dx == -1:
                sh = jnp.where(xc != 0, sh, 0.0)
            if dy == 1:
                sh = jnp.where(yc != (_H - 1), sh, 0.0)
            elif dy == -1:
                sh = jnp.where(yc != 0, sh, 0.0)
            t = (dy + 1) * 3 + (dx + 1)
            acc = acc + _dot_t(sh, w2_ref[t])
    H2 = _gelu(acc)
    sts = []
    for bb in range(_NB):
        h2b = jax.lax.slice(H2, (bb * _P, 0), ((bb + 1) * _P, _MID))
        stb = _dot_t(h2b, w3_ref[...]) + b3_ref[...]
        scores_ref[bb] = jnp.where(en, jnp.transpose(stb, (1, 0)), 0.0)
        sts.append(stb)
    S = jnp.concatenate(sts, axis=1)

    bits = jax.lax.bitcast_convert_type(S, jnp.int32)
    keys = jnp.where(bits >= 0, bits, bits ^ np.int32(0x7FFFFFFF))

    kf = np.float32(_KEEP)
    ones_bf = jnp.ones((1, _P), jnp.bfloat16)

    def _count16(ind_bf):
        return jax.lax.dot_general(ones_bf, ind_bf, (((1,), (0,)), ((), ())),
                                   preferred_element_type=jnp.float32)

    hi16 = (keys >> 16).astype(jnp.int16)
    i16min = np.int32(-32768)

    def step_hi(i, T):
        cand_u = T | (np.int32(1) << (15 - i))
        cand_k = (cand_u ^ i16min).astype(jnp.int16)
        ind = jnp.where(hi16 < cand_k, jnp.bfloat16(1), jnp.bfloat16(0))
        return jnp.where(_count16(ind) < kf, cand_u, T)

    Tu = jax.lax.fori_loop(0, 16, step_hi, jnp.zeros((1, _CW), jnp.int32))
    q = Tu ^ i16min
    q16 = q.astype(jnp.int16)
    g_hi = _count16(jnp.where(hi16 < q16, jnp.bfloat16(1), jnp.bfloat16(0)))
    r2 = kf - g_hi
    lo16 = ((keys & np.int32(0xFFFF)) + i16min).astype(jnp.int16)
    act = jnp.where(hi16 == q16, jnp.bfloat16(1), jnp.bfloat16(0))

    def step_lo(i, T):
        cand_u = T | (np.int32(1) << (15 - i))
        cand_k = (cand_u ^ i16min).astype(jnp.int16)
        ind = jnp.where(lo16 < cand_k, act, jnp.bfloat16(0))
        return jnp.where(_count16(ind) < r2, cand_u, T)

    Tlu = jax.lax.fori_loop(0, 16, step_lo, jnp.zeros((1, _CW), jnp.int32))
    Tk = (q << 16) | (Tlu & np.int32(0xFFFF))

    Tkb = Tk + jnp.zeros((_P, _CW), jnp.int32)
    le = keys <= Tkb
    ones_f = jnp.ones((1, _P), jnp.float32)
    cnt_le = jax.lax.dot_general(ones_f, jnp.where(le, 1.0, 0.0),
                                 (((1,), (0,)), ((), ())),
                                 preferred_element_type=jnp.float32)
    any_tie = jnp.sum(jnp.where(cnt_le != kf, 1, 0)) > 0

    def no_tie():
        return le.astype(jnp.float32)

    def tie_fix():
        lt = keys < Tkb
        eq = keys == Tkb
        g = jnp.sum(lt.astype(jnp.int32), axis=0, keepdims=True)
        c = eq.astype(jnp.int32)
        shift = 1
        while shift < _P:
            c = c + jnp.concatenate(
                [jnp.zeros((shift, _CW), jnp.int32),
                 jax.lax.slice(c, (0, 0), (_P - shift, _CW))], axis=0)
            shift *= 2
        return (lt | (eq & (c <= (_KEEP - g)))).astype(jnp.float32)

    maskT = jax.lax.cond(any_tie, tie_fix, no_tie)
    for bb in range(_NB):
        mb = jax.lax.slice(maskT, (0, bb * _C), (_P, (bb + 1) * _C))
        mask_ref[bb] = jnp.where(en, jnp.transpose(mb, (1, 0)), 1.0)


def _block_diag_w2(w2):
    cpg = _MID // _GROUPS
    take = jnp.take(w2, jnp.asarray(np.arange(_MID) % cpg), axis=1)
    gi = np.arange(_MID) // cpg
    gmask = jnp.asarray((gi[:, None] == gi[None, :]).astype(np.float32))
    full = take * gmask[:, :, None, None]
    return jnp.transpose(full, (2, 3, 0, 1)).reshape(9, _MID, _MID)


def _run(en, xf, w1, b1, w2bd, b2, w3, b3):
    return pl.pallas_call(
        _body,
        grid=(_B // _NB,),
        in_specs=[
            pl.BlockSpec(memory_space=pltpu.SMEM),
            pl.BlockSpec((_NB, _C, _P), lambda b: (b, 0, 0)),
            pl.BlockSpec((_MID, _C), lambda b: (0, 0)),
            pl.BlockSpec((1, _MID), lambda b: (0, 0)),
            pl.BlockSpec((9, _MID, _MID), lambda b: (0, 0, 0)),
            pl.BlockSpec((1, _MID), lambda b: (0, 0)),
            pl.BlockSpec((_C, _MID), lambda b: (0, 0)),
            pl.BlockSpec((1, _C), lambda b: (0, 0)),
        ],
        out_specs=[
            pl.BlockSpec((_NB, _C, _P), lambda b: (b, 0, 0)),
            pl.BlockSpec((_NB, _C, _P), lambda b: (b, 0, 0)),
        ],
        out_shape=[
            jax.ShapeDtypeStruct((_B, _C, _P), jnp.float32),
            jax.ShapeDtypeStruct((_B, _C, _P), jnp.float32),
        ],
    )(en, xf, w1, b1, w2bd, b2, w3, b3)


def kernel(features, enabled, w1, b1, w2, b2, w3, b3):
    xf = features.reshape(_B, _C, _P)
    mask, scores = _run(
        jnp.asarray(enabled, jnp.int32).reshape(1),
        xf,
        w1.reshape(_MID, _C),
        b1.reshape(1, _MID),
        _block_diag_w2(w2),
        b2.reshape(1, _MID),
        w3.reshape(_C, _MID),
        b3.reshape(1, _C),
    )
    return (mask.reshape(_B, _C, _H, _W), scores.reshape(_B, _C, _H, _W))
